# Initial kernel scaffold; baseline (speedup 1.0000x reference)
import jax
import jax.numpy as jnp
from jax.experimental import pallas as pl


def kernel(x, edge_index, W1l, W1r, b1, gamma, beta, W2l, W2r, b2):
    raise NotImplementedError("write your pallas kernel here")


# speedup vs baseline: 5.8137x; 5.8137x over previous
import functools

import jax
import jax.numpy as jnp
from jax import lax
from jax.experimental import pallas as pl
from jax.experimental.pallas import tpu as pltpu
from jax.experimental.pallas import tpu_sc as plsc

N = 10000
E = 320000
D = 128
DEGW = 128
NC = 2
NS = 16
NW = NC * NS
EPW = E // NW
CH = 128
NFULL = EPW // CH
TAIL = EPW - NFULL * CH
RPT = 624
SLC = 640

_mesh = plsc.VectorSubcoreMesh(core_axis_name="c", subcore_axis_name="s")


@functools.partial(
    pl.kernel,
    out_type=jax.ShapeDtypeStruct((NC * N, D), jnp.float32),
    scratch_types=[
        pltpu.VMEM((CH,), jnp.int32),
        pltpu.VMEM((CH,), jnp.int32),
        pltpu.VMEM((CH, D), jnp.float32),
        pltpu.VMEM((TAIL,), jnp.int32),
        pltpu.VMEM((TAIL,), jnp.int32),
        pltpu.VMEM((TAIL, D), jnp.float32),
        pltpu.VMEM_SHARED((N, D), jnp.float32),
        pltpu.SemaphoreType.DMA,
    ],
    mesh=_mesh,
)
def _seg_sum(x_hbm, src_hbm, dst_hbm, zrow_hbm, part_hbm,
             src_v, dst_v, rows_v, src_t, dst_t, rows_t, acc_sh, sem):
  cid = lax.axis_index("c")
  sid = lax.axis_index("s")
  wid = sid * NC + cid
  t0 = sid * RPT

  pltpu.sync_copy(zrow_hbm, acc_sh.at[pl.ds(t0, SLC)])
  plsc.subcore_barrier()

  base = wid * EPW

  @pl.loop(0, NFULL)
  def _edge_loop(g):
    off = base + g * CH
    pltpu.sync_copy(src_hbm.at[pl.ds(off, CH)], src_v)
    pltpu.sync_copy(dst_hbm.at[pl.ds(off, CH)], dst_v)
    pltpu.async_copy(x_hbm.at[src_v], rows_v, sem).wait()
    pltpu.sync_copy(rows_v, acc_sh.at[dst_v], add=True)

  offt = base + NFULL * CH
  pltpu.sync_copy(src_hbm.at[pl.ds(offt, TAIL)], src_t)
  pltpu.sync_copy(dst_hbm.at[pl.ds(offt, TAIL)], dst_t)
  pltpu.async_copy(x_hbm.at[src_t], rows_t, sem).wait()
  pltpu.sync_copy(rows_t, acc_sh.at[dst_t], add=True)

  plsc.subcore_barrier()
  pltpu.sync_copy(acc_sh.at[pl.ds(t0, SLC)],
                  part_hbm.at[pl.ds(cid * N + t0, SLC)])


@functools.partial(
    pl.kernel,
    out_type=jax.ShapeDtypeStruct((NC * N, DEGW), jnp.float32),
    scratch_types=[
        pltpu.VMEM((CH,), jnp.int32),
        pltpu.VMEM((TAIL,), jnp.int32),
        pltpu.VMEM((CH, DEGW), jnp.float32),
        pltpu.VMEM((TAIL, DEGW), jnp.float32),
        pltpu.VMEM_SHARED((N, DEGW), jnp.float32),
        pltpu.SemaphoreType.DMA,
    ],
    mesh=_mesh,
)
def _deg_count(dst_hbm, zdeg_hbm, ones_hbm, degp_hbm,
               dst_v, dst_t, ones_v, ones_t, deg_sh, sem):
  del sem
  cid = lax.axis_index("c")
  sid = lax.axis_index("s")
  wid = sid * NC + cid
  t0 = sid * RPT

  pltpu.sync_copy(zdeg_hbm, deg_sh.at[pl.ds(t0, SLC)])
  pltpu.sync_copy(ones_hbm, ones_v)
  pltpu.sync_copy(ones_hbm.at[pl.ds(0, TAIL)], ones_t)
  plsc.subcore_barrier()

  base = wid * EPW

  @pl.loop(0, NFULL)
  def _edge_loop(g):
    off = base + g * CH
    pltpu.sync_copy(dst_hbm.at[pl.ds(off, CH)], dst_v)
    pltpu.sync_copy(ones_v, deg_sh.at[dst_v], add=True)

  offt = base + NFULL * CH
  pltpu.sync_copy(dst_hbm.at[pl.ds(offt, TAIL)], dst_t)
  pltpu.sync_copy(ones_t, deg_sh.at[dst_t], add=True)

  plsc.subcore_barrier()
  pltpu.sync_copy(deg_sh.at[pl.ds(t0, SLC)],
                  degp_hbm.at[pl.ds(cid * N + t0, SLC)])


def _mm1_body(p_ref, degp_ref, x_ref, wl_ref, wr_ref, b_ref,
              z_ref, s_ref, ss_ref):
  s = p_ref[0] + p_ref[1]
  deg = degp_ref[0, :, 0:1] + degp_ref[1, :, 0:1]
  agg = s / jnp.maximum(deg, 1.0)
  z = (lax.dot_general(agg, wl_ref[...], (((1,), (1,)), ((), ())),
                       preferred_element_type=jnp.float32)
       + lax.dot_general(x_ref[...], wr_ref[...], (((1,), (1,)), ((), ())),
                         preferred_element_type=jnp.float32)
       + b_ref[...])
  z_ref[...] = z
  s_ref[...] = jnp.broadcast_to(jnp.sum(z, axis=0, keepdims=True), (8, D))
  ss_ref[...] = jnp.broadcast_to(jnp.sum(z * z, axis=0, keepdims=True), (8, D))


_mm1 = pl.pallas_call(
    _mm1_body,
    out_shape=[
        jax.ShapeDtypeStruct((N, D), jnp.float32),
        jax.ShapeDtypeStruct((8, D), jnp.float32),
        jax.ShapeDtypeStruct((8, D), jnp.float32),
    ],
)


def _bn_relu_body(z_ref, s_ref, ss_ref, g_ref, bt_ref, h_ref):
  mean = s_ref[0:1] * (1.0 / N)
  var = ss_ref[0:1] * (1.0 / N) - mean * mean
  inv = lax.rsqrt(var + 1e-5)
  h = (z_ref[...] - mean) * (inv * g_ref[...]) + bt_ref[...]
  h_ref[...] = jnp.maximum(h, 0.0)


_bn_relu = pl.pallas_call(
    _bn_relu_body,
    out_shape=jax.ShapeDtypeStruct((N, D), jnp.float32),
)


def _mm2_body(p_ref, degp_ref, h_ref, wl_ref, wr_ref, b_ref, o_ref):
  s = p_ref[0] + p_ref[1]
  deg = degp_ref[0, :, 0:1] + degp_ref[1, :, 0:1]
  agg = s / jnp.maximum(deg, 1.0)
  o_ref[...] = (lax.dot_general(agg, wl_ref[...], (((1,), (1,)), ((), ())),
                                preferred_element_type=jnp.float32)
                + lax.dot_general(h_ref[...], wr_ref[...], (((1,), (1,)), ((), ())),
                                  preferred_element_type=jnp.float32)
                + b_ref[...])


_mm2 = pl.pallas_call(
    _mm2_body,
    out_shape=jax.ShapeDtypeStruct((N, D), jnp.float32),
)


@functools.partial(
    pl.kernel,
    out_type=jax.ShapeDtypeStruct((256, D), jnp.float32),
    scratch_types=[
        pltpu.VMEM((8,), jnp.int32),
        pltpu.VMEM((8,), jnp.int32),
        pltpu.VMEM((8, D), jnp.float32),
        pltpu.VMEM_SHARED((128, D), jnp.float32),
        pltpu.SemaphoreType.DMA,
    ],
    mesh=_mesh,
)
def _sc_probe(x_hbm, idx_hbm, dst_hbm, zrow_hbm, out_hbm, idx_v, dst_v, rows_v,
              acc_sh, sem):
  cid = lax.axis_index("c")
  sid = lax.axis_index("s")
  wid = sid * NC + cid
  pltpu.sync_copy(zrow_hbm, acc_sh.at[pl.ds(sid * 8, 8)])
  plsc.subcore_barrier()

  @pl.loop(0, 2)
  def _lp(g):
    base = wid * 8 + g * 256
    pltpu.sync_copy(idx_hbm.at[pl.ds(base, 8)], idx_v)
    pltpu.sync_copy(dst_hbm.at[pl.ds(base, 8)], dst_v)
    pltpu.async_copy(x_hbm.at[idx_v], rows_v, sem).wait()
    pltpu.sync_copy(rows_v, acc_sh.at[dst_v], add=True)

  plsc.subcore_barrier()
  pltpu.sync_copy(acc_sh.at[pl.ds(sid * 8, 8)],
                  out_hbm.at[pl.ds(cid * 128 + sid * 8, 8)])


def kernel(x, edge_index, W1l, W1r, b1, gamma, beta, W2l, W2r, b2):
  src = edge_index[0].astype(jnp.int32)
  dst = edge_index[1].astype(jnp.int32)

  zrow = jnp.zeros((SLC, D), jnp.float32)
  ones = jnp.zeros((CH, DEGW), jnp.float32).at[:, 0].set(1.0)

  degp = _deg_count(dst, zrow, ones).reshape(NC, N, DEGW)
  part1 = _seg_sum(x, src, dst, zrow).reshape(NC, N, D)
  z, s, ss = _mm1(part1, degp, x, W1l, W1r, b1.reshape(1, D))
  h = _bn_relu(z, s, ss, gamma.reshape(1, D), beta.reshape(1, D))
  part2 = _seg_sum(h, src, dst, zrow).reshape(NC, N, D)
  out = _mm2(part2, degp, h, W2l, W2r, b2.reshape(1, D))
  return out


# speedup vs baseline: 8.1972x; 1.4100x over previous
import functools

import jax
import jax.numpy as jnp
from jax import lax
from jax.experimental import pallas as pl
from jax.experimental.pallas import tpu as pltpu
from jax.experimental.pallas import tpu_sc as plsc

N = 10000
E = 320000
D = 128
DEGW = 128
NC = 2
NS = 16
NW = NC * NS
EPW = E // NW
CH = 128
NFULL = EPW // CH
TAIL = EPW - NFULL * CH
RPT = 624
SLC = 640

_mesh = plsc.VectorSubcoreMesh(core_axis_name="c", subcore_axis_name="s")


@functools.partial(
    pl.kernel,
    out_type=jax.ShapeDtypeStruct((NC * N, D), jnp.float32),
    scratch_types=[
        pltpu.VMEM((2, CH), jnp.int32),
        pltpu.VMEM((2, CH), jnp.int32),
        pltpu.VMEM((2, CH, D), jnp.float32),
        pltpu.VMEM((TAIL,), jnp.int32),
        pltpu.VMEM((TAIL,), jnp.int32),
        pltpu.VMEM((TAIL, D), jnp.float32),
        pltpu.VMEM_SHARED((N, D), jnp.float32),
        pltpu.SemaphoreType.DMA((2,)),
    ],
    mesh=_mesh,
)
def _seg_sum(x_hbm, src_hbm, dst_hbm, zrow_hbm, part_hbm,
             src2, dst2, rows2, src_t, dst_t, rows_t, acc_sh, sem):
  cid = lax.axis_index("c")
  sid = lax.axis_index("s")
  wid = sid * NC + cid
  t0 = sid * RPT

  pltpu.sync_copy(zrow_hbm, acc_sh.at[pl.ds(t0, SLC)])
  plsc.subcore_barrier()

  base = wid * EPW

  def _start_gather(c, b):
    off = base + c * CH
    pltpu.sync_copy(src_hbm.at[pl.ds(off, CH)], src2.at[b])
    pltpu.sync_copy(dst_hbm.at[pl.ds(off, CH)], dst2.at[b])
    pltpu.async_copy(x_hbm.at[src2.at[b]], rows2.at[b], sem.at[b])

  def _finish_chunk(b):
    pltpu.make_async_copy(x_hbm.at[src2.at[b]], rows2.at[b], sem.at[b]).wait()
    pltpu.sync_copy(rows2.at[b], acc_sh.at[dst2.at[b]], add=True)

  _start_gather(0, 0)

  @pl.loop(0, NFULL // 2 - 1)
  def _edge_loop(g2):
    for b in (0, 1):
      c = g2 * 2 + b
      _start_gather(c + 1, b ^ 1)
      _finish_chunk(b)

  _start_gather(NFULL - 1, 1)
  _finish_chunk(0)
  _finish_chunk(1)

  offt = base + NFULL * CH
  pltpu.sync_copy(src_hbm.at[pl.ds(offt, TAIL)], src_t)
  pltpu.sync_copy(dst_hbm.at[pl.ds(offt, TAIL)], dst_t)
  pltpu.async_copy(x_hbm.at[src_t], rows_t, sem.at[0]).wait()
  pltpu.sync_copy(rows_t, acc_sh.at[dst_t], add=True)

  plsc.subcore_barrier()
  pltpu.sync_copy(acc_sh.at[pl.ds(t0, SLC)],
                  part_hbm.at[pl.ds(cid * N + t0, SLC)])


@functools.partial(
    pl.kernel,
    out_type=jax.ShapeDtypeStruct((NC * N, DEGW), jnp.float32),
    scratch_types=[
        pltpu.VMEM((CH,), jnp.int32),
        pltpu.VMEM((TAIL,), jnp.int32),
        pltpu.VMEM((CH, DEGW), jnp.float32),
        pltpu.VMEM((TAIL, DEGW), jnp.float32),
        pltpu.VMEM_SHARED((N, DEGW), jnp.float32),
        pltpu.SemaphoreType.DMA,
    ],
    mesh=_mesh,
)
def _deg_count(dst_hbm, zdeg_hbm, ones_hbm, degp_hbm,
               dst_v, dst_t, ones_v, ones_t, deg_sh, sem):
  del sem
  cid = lax.axis_index("c")
  sid = lax.axis_index("s")
  wid = sid * NC + cid
  t0 = sid * RPT

  pltpu.sync_copy(zdeg_hbm, deg_sh.at[pl.ds(t0, SLC)])
  pltpu.sync_copy(ones_hbm, ones_v)
  pltpu.sync_copy(ones_hbm.at[pl.ds(0, TAIL)], ones_t)
  plsc.subcore_barrier()

  base = wid * EPW

  @pl.loop(0, NFULL)
  def _edge_loop(g):
    off = base + g * CH
    pltpu.sync_copy(dst_hbm.at[pl.ds(off, CH)], dst_v)
    pltpu.sync_copy(ones_v, deg_sh.at[dst_v], add=True)

  offt = base + NFULL * CH
  pltpu.sync_copy(dst_hbm.at[pl.ds(offt, TAIL)], dst_t)
  pltpu.sync_copy(ones_t, deg_sh.at[dst_t], add=True)

  plsc.subcore_barrier()
  pltpu.sync_copy(deg_sh.at[pl.ds(t0, SLC)],
                  degp_hbm.at[pl.ds(cid * N + t0, SLC)])


def _mm1_body(p_ref, degp_ref, x_ref, wl_ref, wr_ref, b_ref,
              z_ref, s_ref, ss_ref):
  s = p_ref[0] + p_ref[1]
  deg = degp_ref[0, :, 0:1] + degp_ref[1, :, 0:1]
  agg = s / jnp.maximum(deg, 1.0)
  z = (lax.dot_general(agg, wl_ref[...], (((1,), (1,)), ((), ())),
                       preferred_element_type=jnp.float32)
       + lax.dot_general(x_ref[...], wr_ref[...], (((1,), (1,)), ((), ())),
                         preferred_element_type=jnp.float32)
       + b_ref[...])
  z_ref[...] = z
  s_ref[...] = jnp.broadcast_to(jnp.sum(z, axis=0, keepdims=True), (8, D))
  ss_ref[...] = jnp.broadcast_to(jnp.sum(z * z, axis=0, keepdims=True), (8, D))


_mm1 = pl.pallas_call(
    _mm1_body,
    out_shape=[
        jax.ShapeDtypeStruct((N, D), jnp.float32),
        jax.ShapeDtypeStruct((8, D), jnp.float32),
        jax.ShapeDtypeStruct((8, D), jnp.float32),
    ],
)


def _bn_relu_body(z_ref, s_ref, ss_ref, g_ref, bt_ref, h_ref):
  mean = s_ref[0:1] * (1.0 / N)
  var = ss_ref[0:1] * (1.0 / N) - mean * mean
  inv = lax.rsqrt(var + 1e-5)
  h = (z_ref[...] - mean) * (inv * g_ref[...]) + bt_ref[...]
  h_ref[...] = jnp.maximum(h, 0.0)


_bn_relu = pl.pallas_call(
    _bn_relu_body,
    out_shape=jax.ShapeDtypeStruct((N, D), jnp.float32),
)


def _mm2_body(p_ref, degp_ref, h_ref, wl_ref, wr_ref, b_ref, o_ref):
  s = p_ref[0] + p_ref[1]
  deg = degp_ref[0, :, 0:1] + degp_ref[1, :, 0:1]
  agg = s / jnp.maximum(deg, 1.0)
  o_ref[...] = (lax.dot_general(agg, wl_ref[...], (((1,), (1,)), ((), ())),
                                preferred_element_type=jnp.float32)
                + lax.dot_general(h_ref[...], wr_ref[...], (((1,), (1,)), ((), ())),
                                  preferred_element_type=jnp.float32)
                + b_ref[...])


_mm2 = pl.pallas_call(
    _mm2_body,
    out_shape=jax.ShapeDtypeStruct((N, D), jnp.float32),
)


@functools.partial(
    pl.kernel,
    out_type=jax.ShapeDtypeStruct((256, D), jnp.float32),
    scratch_types=[
        pltpu.VMEM((8,), jnp.int32),
        pltpu.VMEM((8,), jnp.int32),
        pltpu.VMEM((8, D), jnp.float32),
        pltpu.VMEM_SHARED((128, D), jnp.float32),
        pltpu.SemaphoreType.DMA,
    ],
    mesh=_mesh,
)
def _sc_probe(x_hbm, idx_hbm, dst_hbm, zrow_hbm, out_hbm, idx_v, dst_v, rows_v,
              acc_sh, sem):
  cid = lax.axis_index("c")
  sid = lax.axis_index("s")
  wid = sid * NC + cid
  pltpu.sync_copy(zrow_hbm, acc_sh.at[pl.ds(sid * 8, 8)])
  plsc.subcore_barrier()

  @pl.loop(0, 2)
  def _lp(g):
    base = wid * 8 + g * 256
    pltpu.sync_copy(idx_hbm.at[pl.ds(base, 8)], idx_v)
    pltpu.sync_copy(dst_hbm.at[pl.ds(base, 8)], dst_v)
    pltpu.async_copy(x_hbm.at[idx_v], rows_v, sem).wait()
    pltpu.sync_copy(rows_v, acc_sh.at[dst_v], add=True)

  plsc.subcore_barrier()
  pltpu.sync_copy(acc_sh.at[pl.ds(sid * 8, 8)],
                  out_hbm.at[pl.ds(cid * 128 + sid * 8, 8)])


def kernel(x, edge_index, W1l, W1r, b1, gamma, beta, W2l, W2r, b2):
  src = edge_index[0].astype(jnp.int32)
  dst = edge_index[1].astype(jnp.int32)

  zrow = jnp.zeros((SLC, D), jnp.float32)
  ones = jnp.zeros((CH, DEGW), jnp.float32).at[:, 0].set(1.0)

  degp = _deg_count(dst, zrow, ones).reshape(NC, N, DEGW)
  part1 = _seg_sum(x, src, dst, zrow).reshape(NC, N, D)
  z, s, ss = _mm1(part1, degp, x, W1l, W1r, b1.reshape(1, D))
  h = _bn_relu(z, s, ss, gamma.reshape(1, D), beta.reshape(1, D))
  part2 = _seg_sum(h, src, dst, zrow).reshape(NC, N, D)
  out = _mm2(part2, degp, h, W2l, W2r, b2.reshape(1, D))
  return out


# speedup vs baseline: 9.5141x; 1.1606x over previous
import functools

import jax
import jax.numpy as jnp
from jax import lax
from jax.experimental import pallas as pl
from jax.experimental.pallas import tpu as pltpu
from jax.experimental.pallas import tpu_sc as plsc

N = 10000
E = 320000
D = 128
NC = 2
NS = 16
NW = NC * NS
EPW = E // NW
CH = 128
NFULL = EPW // CH
TAIL = EPW - NFULL * CH
RPT = 624
SLC = 640

_mesh = plsc.VectorSubcoreMesh(core_axis_name="c", subcore_axis_name="s")


@functools.partial(
    pl.kernel,
    out_type=jax.ShapeDtypeStruct((NC * N, D), jnp.float32),
    scratch_types=[
        pltpu.VMEM((2, CH), jnp.int32),
        pltpu.VMEM((2, CH), jnp.int32),
        pltpu.VMEM((2, CH, D), jnp.float32),
        pltpu.VMEM((TAIL,), jnp.int32),
        pltpu.VMEM((TAIL,), jnp.int32),
        pltpu.VMEM((TAIL, D), jnp.float32),
        pltpu.VMEM_SHARED((N, D), jnp.float32),
        pltpu.SemaphoreType.DMA((2,)),
    ],
    mesh=_mesh,
)
def _seg_sum(x_hbm, src_hbm, dst_hbm, zrow_hbm, part_hbm,
             src2, dst2, rows2, src_t, dst_t, rows_t, acc_sh, sem):
  cid = lax.axis_index("c")
  sid = lax.axis_index("s")
  wid = sid * NC + cid
  t0 = sid * RPT

  pltpu.sync_copy(zrow_hbm, acc_sh.at[pl.ds(t0, SLC)])
  plsc.subcore_barrier()

  base = wid * EPW

  def _start_gather(c, b):
    off = base + c * CH
    pltpu.sync_copy(src_hbm.at[pl.ds(off, CH)], src2.at[b])
    pltpu.sync_copy(dst_hbm.at[pl.ds(off, CH)], dst2.at[b])
    pltpu.async_copy(x_hbm.at[src2.at[b]], rows2.at[b], sem.at[b])

  def _finish_chunk(b):
    pltpu.make_async_copy(x_hbm.at[src2.at[b]], rows2.at[b], sem.at[b]).wait()
    pltpu.sync_copy(rows2.at[b], acc_sh.at[dst2.at[b]], add=True)

  _start_gather(0, 0)

  @pl.loop(0, NFULL // 2 - 1)
  def _edge_loop(g2):
    for b in (0, 1):
      c = g2 * 2 + b
      _start_gather(c + 1, b ^ 1)
      _finish_chunk(b)

  _start_gather(NFULL - 1, 1)
  _finish_chunk(0)
  _finish_chunk(1)

  offt = base + NFULL * CH
  pltpu.sync_copy(src_hbm.at[pl.ds(offt, TAIL)], src_t)
  pltpu.sync_copy(dst_hbm.at[pl.ds(offt, TAIL)], dst_t)
  pltpu.async_copy(x_hbm.at[src_t], rows_t, sem.at[0]).wait()
  pltpu.sync_copy(rows_t, acc_sh.at[dst_t], add=True)

  plsc.subcore_barrier()
  pltpu.sync_copy(acc_sh.at[pl.ds(t0, SLC)],
                  part_hbm.at[pl.ds(cid * N + t0, SLC)])


NPAD = 10240


@functools.partial(
    pl.kernel,
    out_type=jax.ShapeDtypeStruct((NW * NPAD,), jnp.float32),
    scratch_types=[
        pltpu.VMEM((CH,), jnp.int32),
        pltpu.VMEM((TAIL,), jnp.int32),
        pltpu.VMEM((NPAD,), jnp.float32),
    ],
    mesh=_mesh,
    compiler_params=pltpu.CompilerParams(needs_layout_passes=False),
)
def _deg_hist(dst_hbm, degp_hbm, dst_v, dst_t, degloc):
  cid = lax.axis_index("c")
  sid = lax.axis_index("s")
  wid = sid * NC + cid

  zero16 = jnp.zeros((16,), jnp.float32)

  @pl.loop(0, NPAD // 16)
  def _zero(i):
    degloc[pl.ds(i * 16, 16)] = zero16

  base = wid * EPW
  ones16 = jnp.ones((16,), jnp.float32)

  @pl.loop(0, NFULL)
  def _chunk(g):
    off = base + g * CH
    pltpu.sync_copy(dst_hbm.at[pl.ds(off, CH)], dst_v)
    for j in range(CH // 16):
      idx = dst_v[pl.ds(j * 16, 16)]
      plsc.addupdate_scatter(degloc, [idx], ones16)

  offt = base + NFULL * CH
  pltpu.sync_copy(dst_hbm.at[pl.ds(offt, TAIL)], dst_t)
  plsc.addupdate_scatter(degloc, [dst_t[...]], ones16)

  pltpu.sync_copy(degloc, degp_hbm.at[pl.ds(wid * NPAD, NPAD)])


def _mm1_body(p_ref, degt_ref, x_ref, wl_ref, wr_ref, b_ref,
              z_ref, s_ref, ss_ref):
  s = p_ref[0] + p_ref[1]
  deg = jnp.sum(degt_ref[...], axis=1, keepdims=True)
  agg = s / jnp.maximum(deg, 1.0)
  z = (lax.dot_general(agg, wl_ref[...], (((1,), (1,)), ((), ())),
                       preferred_element_type=jnp.float32)
       + lax.dot_general(x_ref[...], wr_ref[...], (((1,), (1,)), ((), ())),
                         preferred_element_type=jnp.float32)
       + b_ref[...])
  z_ref[...] = z
  s_ref[...] = jnp.broadcast_to(jnp.sum(z, axis=0, keepdims=True), (8, D))
  ss_ref[...] = jnp.broadcast_to(jnp.sum(z * z, axis=0, keepdims=True), (8, D))


_mm1 = pl.pallas_call(
    _mm1_body,
    out_shape=[
        jax.ShapeDtypeStruct((N, D), jnp.float32),
        jax.ShapeDtypeStruct((8, D), jnp.float32),
        jax.ShapeDtypeStruct((8, D), jnp.float32),
    ],
)


def _bn_relu_body(z_ref, s_ref, ss_ref, g_ref, bt_ref, h_ref):
  mean = s_ref[0:1] * (1.0 / N)
  var = ss_ref[0:1] * (1.0 / N) - mean * mean
  inv = lax.rsqrt(var + 1e-5)
  h = (z_ref[...] - mean) * (inv * g_ref[...]) + bt_ref[...]
  h_ref[...] = jnp.maximum(h, 0.0)


_bn_relu = pl.pallas_call(
    _bn_relu_body,
    out_shape=jax.ShapeDtypeStruct((N, D), jnp.float32),
)


def _mm2_body(p_ref, degt_ref, h_ref, wl_ref, wr_ref, b_ref, o_ref):
  s = p_ref[0] + p_ref[1]
  deg = jnp.sum(degt_ref[...], axis=1, keepdims=True)
  agg = s / jnp.maximum(deg, 1.0)
  o_ref[...] = (lax.dot_general(agg, wl_ref[...], (((1,), (1,)), ((), ())),
                                preferred_element_type=jnp.float32)
                + lax.dot_general(h_ref[...], wr_ref[...], (((1,), (1,)), ((), ())),
                                  preferred_element_type=jnp.float32)
                + b_ref[...])


_mm2 = pl.pallas_call(
    _mm2_body,
    out_shape=jax.ShapeDtypeStruct((N, D), jnp.float32),
)


def kernel(x, edge_index, W1l, W1r, b1, gamma, beta, W2l, W2r, b2):
  src = edge_index[0].astype(jnp.int32)
  dst = edge_index[1].astype(jnp.int32)

  zrow = jnp.zeros((SLC, D), jnp.float32)

  degt = _deg_hist(dst).reshape(NW, NPAD)[:, :N].T
  part1 = _seg_sum(x, src, dst, zrow).reshape(NC, N, D)
  z, s, ss = _mm1(part1, degt, x, W1l, W1r, b1.reshape(1, D))
  h = _bn_relu(z, s, ss, gamma.reshape(1, D), beta.reshape(1, D))
  part2 = _seg_sum(h, src, dst, zrow).reshape(NC, N, D)
  out = _mm2(part2, degt, h, W2l, W2r, b2.reshape(1, D))
  return out


# speedup vs baseline: 10.9208x; 1.1479x over previous
import functools

import jax
import jax.numpy as jnp
from jax import lax
from jax.experimental import pallas as pl
from jax.experimental.pallas import tpu as pltpu
from jax.experimental.pallas import tpu_sc as plsc

N = 10000
E = 320000
D = 128
NC = 2
NS = 16
NW = NC * NS
EPW = E // NW
CH = 128
NFULL = EPW // CH
TAIL = EPW - NFULL * CH
RPT = 624
SLC = 640

_mesh = plsc.VectorSubcoreMesh(core_axis_name="c", subcore_axis_name="s")


@functools.partial(
    pl.kernel,
    out_type=jax.ShapeDtypeStruct((NC * N, D), jnp.float32),
    scratch_types=[
        pltpu.VMEM((2, CH), jnp.int32),
        pltpu.VMEM((2, CH), jnp.int32),
        pltpu.VMEM((2, CH, D), jnp.float32),
        pltpu.VMEM((TAIL,), jnp.int32),
        pltpu.VMEM((TAIL,), jnp.int32),
        pltpu.VMEM((TAIL, D), jnp.float32),
        pltpu.VMEM_SHARED((N, D), jnp.float32),
        pltpu.SemaphoreType.DMA((6,)),
    ],
    mesh=_mesh,
)
def _seg_sum(x_hbm, src_hbm, dst_hbm, zrow_hbm, part_hbm,
             src2, dst2, rows2, src_t, dst_t, rows_t, acc_sh, sem):
  cid = lax.axis_index("c")
  sid = lax.axis_index("s")
  wid = sid * NC + cid
  t0 = sid * RPT

  pltpu.sync_copy(zrow_hbm, acc_sh.at[pl.ds(t0, SLC)])
  plsc.subcore_barrier()

  base = wid * EPW

  def _load_idx(c, b):
    off = base + c * CH
    pltpu.async_copy(src_hbm.at[pl.ds(off, CH)], src2.at[b], sem.at[2 + b])
    pltpu.async_copy(dst_hbm.at[pl.ds(off, CH)], dst2.at[b], sem.at[4 + b])

  def _wait_idx(c, b):
    off = base + c * CH
    pltpu.make_async_copy(src_hbm.at[pl.ds(off, CH)], src2.at[b],
                          sem.at[2 + b]).wait()
    pltpu.make_async_copy(dst_hbm.at[pl.ds(off, CH)], dst2.at[b],
                          sem.at[4 + b]).wait()

  def _start_gather(b):
    pltpu.async_copy(x_hbm.at[src2.at[b]], rows2.at[b], sem.at[b])

  def _finish_chunk(b):
    pltpu.make_async_copy(x_hbm.at[src2.at[b]], rows2.at[b], sem.at[b]).wait()
    pltpu.sync_copy(rows2.at[b], acc_sh.at[dst2.at[b]], add=True)

  _load_idx(0, 0)
  _wait_idx(0, 0)
  _start_gather(0)
  _load_idx(1, 1)

  @pl.loop(0, (NFULL - 2) // 2)
  def _edge_loop(g2):
    for b in (0, 1):
      c = g2 * 2 + b
      _wait_idx(c + 1, b ^ 1)
      _start_gather(b ^ 1)
      _finish_chunk(b)
      _load_idx(c + 2, b)

  _wait_idx(NFULL - 1, 1)
  _start_gather(1)
  _finish_chunk(0)
  _finish_chunk(1)

  offt = base + NFULL * CH
  pltpu.sync_copy(src_hbm.at[pl.ds(offt, TAIL)], src_t)
  pltpu.sync_copy(dst_hbm.at[pl.ds(offt, TAIL)], dst_t)
  pltpu.async_copy(x_hbm.at[src_t], rows_t, sem.at[0]).wait()
  pltpu.sync_copy(rows_t, acc_sh.at[dst_t], add=True)

  plsc.subcore_barrier()
  pltpu.sync_copy(acc_sh.at[pl.ds(t0, SLC)],
                  part_hbm.at[pl.ds(cid * N + t0, SLC)])


NPAD = 10240


@functools.partial(
    pl.kernel,
    out_type=jax.ShapeDtypeStruct((NW * NPAD,), jnp.float32),
    scratch_types=[
        pltpu.VMEM((CH,), jnp.int32),
        pltpu.VMEM((TAIL,), jnp.int32),
        pltpu.VMEM((NPAD,), jnp.float32),
    ],
    mesh=_mesh,
    compiler_params=pltpu.CompilerParams(needs_layout_passes=False),
)
def _deg_hist(dst_hbm, degp_hbm, dst_v, dst_t, degloc):
  cid = lax.axis_index("c")
  sid = lax.axis_index("s")
  wid = sid * NC + cid

  zero16 = jnp.zeros((16,), jnp.float32)

  @pl.loop(0, NPAD // 16)
  def _zero(i):
    degloc[pl.ds(i * 16, 16)] = zero16

  base = wid * EPW
  ones16 = jnp.ones((16,), jnp.float32)

  @pl.loop(0, NFULL)
  def _chunk(g):
    off = base + g * CH
    pltpu.sync_copy(dst_hbm.at[pl.ds(off, CH)], dst_v)
    for j in range(CH // 16):
      idx = dst_v[pl.ds(j * 16, 16)]
      plsc.addupdate_scatter(degloc, [idx], ones16)

  offt = base + NFULL * CH
  pltpu.sync_copy(dst_hbm.at[pl.ds(offt, TAIL)], dst_t)
  plsc.addupdate_scatter(degloc, [dst_t[...]], ones16)

  pltpu.sync_copy(degloc, degp_hbm.at[pl.ds(wid * NPAD, NPAD)])


def _mm1_body(p_ref, degt_ref, x_ref, wl_ref, wr_ref, b_ref,
              z_ref, s_ref, ss_ref):
  s = p_ref[0] + p_ref[1]
  deg = jnp.sum(degt_ref[...], axis=1, keepdims=True)
  agg = s / jnp.maximum(deg, 1.0)
  z = (lax.dot_general(agg, wl_ref[...], (((1,), (1,)), ((), ())),
                       preferred_element_type=jnp.float32)
       + lax.dot_general(x_ref[...], wr_ref[...], (((1,), (1,)), ((), ())),
                         preferred_element_type=jnp.float32)
       + b_ref[...])
  z_ref[...] = z
  s_ref[...] = jnp.broadcast_to(jnp.sum(z, axis=0, keepdims=True), (8, D))
  ss_ref[...] = jnp.broadcast_to(jnp.sum(z * z, axis=0, keepdims=True), (8, D))


_mm1 = pl.pallas_call(
    _mm1_body,
    out_shape=[
        jax.ShapeDtypeStruct((N, D), jnp.float32),
        jax.ShapeDtypeStruct((8, D), jnp.float32),
        jax.ShapeDtypeStruct((8, D), jnp.float32),
    ],
)


def _bn_relu_body(z_ref, s_ref, ss_ref, g_ref, bt_ref, h_ref):
  mean = s_ref[0:1] * (1.0 / N)
  var = ss_ref[0:1] * (1.0 / N) - mean * mean
  inv = lax.rsqrt(var + 1e-5)
  h = (z_ref[...] - mean) * (inv * g_ref[...]) + bt_ref[...]
  h_ref[...] = jnp.maximum(h, 0.0)


_bn_relu = pl.pallas_call(
    _bn_relu_body,
    out_shape=jax.ShapeDtypeStruct((N, D), jnp.float32),
)


def _mm2_body(p_ref, degt_ref, h_ref, wl_ref, wr_ref, b_ref, o_ref):
  s = p_ref[0] + p_ref[1]
  deg = jnp.sum(degt_ref[...], axis=1, keepdims=True)
  agg = s / jnp.maximum(deg, 1.0)
  o_ref[...] = (lax.dot_general(agg, wl_ref[...], (((1,), (1,)), ((), ())),
                                preferred_element_type=jnp.float32)
                + lax.dot_general(h_ref[...], wr_ref[...], (((1,), (1,)), ((), ())),
                                  preferred_element_type=jnp.float32)
                + b_ref[...])


_mm2 = pl.pallas_call(
    _mm2_body,
    out_shape=jax.ShapeDtypeStruct((N, D), jnp.float32),
)


def kernel(x, edge_index, W1l, W1r, b1, gamma, beta, W2l, W2r, b2):
  src = edge_index[0].astype(jnp.int32)
  dst = edge_index[1].astype(jnp.int32)

  zrow = jnp.zeros((SLC, D), jnp.float32)

  degt = _deg_hist(dst).reshape(NW, NPAD)[:, :N].T
  part1 = _seg_sum(x, src, dst, zrow).reshape(NC, N, D)
  z, s, ss = _mm1(part1, degt, x, W1l, W1r, b1.reshape(1, D))
  h = _bn_relu(z, s, ss, gamma.reshape(1, D), beta.reshape(1, D))
  part2 = _seg_sum(h, src, dst, zrow).reshape(NC, N, D)
  out = _mm2(part2, degt, h, W2l, W2r, b2.reshape(1, D))
  return out


# speedup vs baseline: 11.5755x; 1.0599x over previous
import functools

import jax
import jax.numpy as jnp
from jax import lax
from jax.experimental import pallas as pl
from jax.experimental.pallas import tpu as pltpu
from jax.experimental.pallas import tpu_sc as plsc

N = 10000
E = 320000
D = 128
NC = 2
NS = 16
NW = NC * NS
EPW = E // NW
CH = 128
NFULL = EPW // CH
TAIL = EPW - NFULL * CH
RPT = 624
SLC = 640

_mesh = plsc.VectorSubcoreMesh(core_axis_name="c", subcore_axis_name="s")


@functools.partial(
    pl.kernel,
    out_type=jax.ShapeDtypeStruct((NC * N, D), jnp.float32),
    scratch_types=[
        pltpu.VMEM((2, CH), jnp.int32),
        pltpu.VMEM((2, CH), jnp.int32),
        pltpu.VMEM((2, CH, D), jnp.float32),
        pltpu.VMEM((TAIL,), jnp.int32),
        pltpu.VMEM((TAIL,), jnp.int32),
        pltpu.VMEM((TAIL, D), jnp.float32),
        pltpu.VMEM_SHARED((N, D), jnp.float32),
        pltpu.SemaphoreType.DMA((6,)),
    ],
    mesh=_mesh,
)
def _seg_sum(x_hbm, src_hbm, dst_hbm, zrow_hbm, part_hbm,
             src2, dst2, rows2, src_t, dst_t, rows_t, acc_sh, sem):
  cid = lax.axis_index("c")
  sid = lax.axis_index("s")
  wid = sid * NC + cid
  t0 = sid * RPT

  pltpu.sync_copy(zrow_hbm, acc_sh.at[pl.ds(t0, SLC)])
  plsc.subcore_barrier()

  base = wid * EPW

  def _load_idx(c, b):
    off = base + c * CH
    pltpu.async_copy(src_hbm.at[pl.ds(off, CH)], src2.at[b], sem.at[2 + b])
    pltpu.async_copy(dst_hbm.at[pl.ds(off, CH)], dst2.at[b], sem.at[4 + b])

  def _wait_idx(c, b):
    off = base + c * CH
    pltpu.make_async_copy(src_hbm.at[pl.ds(off, CH)], src2.at[b],
                          sem.at[2 + b]).wait()
    pltpu.make_async_copy(dst_hbm.at[pl.ds(off, CH)], dst2.at[b],
                          sem.at[4 + b]).wait()

  def _start_gather(b):
    pltpu.async_copy(x_hbm.at[src2.at[b]], rows2.at[b], sem.at[b])

  def _finish_chunk(b):
    pltpu.make_async_copy(x_hbm.at[src2.at[b]], rows2.at[b], sem.at[b]).wait()
    pltpu.sync_copy(rows2.at[b], acc_sh.at[dst2.at[b]], add=True)

  _load_idx(0, 0)
  _wait_idx(0, 0)
  _start_gather(0)
  _load_idx(1, 1)

  @pl.loop(0, (NFULL - 2) // 2)
  def _edge_loop(g2):
    for b in (0, 1):
      c = g2 * 2 + b
      _wait_idx(c + 1, b ^ 1)
      _start_gather(b ^ 1)
      _finish_chunk(b)
      _load_idx(c + 2, b)

  _wait_idx(NFULL - 1, 1)
  _start_gather(1)
  _finish_chunk(0)
  _finish_chunk(1)

  offt = base + NFULL * CH
  pltpu.sync_copy(src_hbm.at[pl.ds(offt, TAIL)], src_t)
  pltpu.sync_copy(dst_hbm.at[pl.ds(offt, TAIL)], dst_t)
  pltpu.async_copy(x_hbm.at[src_t], rows_t, sem.at[0]).wait()
  pltpu.sync_copy(rows_t, acc_sh.at[dst_t], add=True)

  plsc.subcore_barrier()
  pltpu.sync_copy(acc_sh.at[pl.ds(t0, SLC)],
                  part_hbm.at[pl.ds(cid * N + t0, SLC)])


NPAD = 10240


@functools.partial(
    pl.kernel,
    out_type=jax.ShapeDtypeStruct((NW * NPAD,), jnp.float32),
    scratch_types=[
        pltpu.VMEM((2, CH), jnp.int32),
        pltpu.VMEM((TAIL,), jnp.int32),
        pltpu.VMEM((NPAD,), jnp.float32),
        pltpu.SemaphoreType.DMA((2,)),
    ],
    mesh=_mesh,
    compiler_params=pltpu.CompilerParams(needs_layout_passes=False),
)
def _deg_hist(dst_hbm, degp_hbm, dst2, dst_t, degloc, sem):
  cid = lax.axis_index("c")
  sid = lax.axis_index("s")
  wid = sid * NC + cid
  base = wid * EPW

  def _load(c, b):
    pltpu.async_copy(dst_hbm.at[pl.ds(base + c * CH, CH)], dst2.at[b],
                     sem.at[b])

  def _wait(c, b):
    pltpu.make_async_copy(dst_hbm.at[pl.ds(base + c * CH, CH)], dst2.at[b],
                          sem.at[b]).wait()

  _load(0, 0)

  zero16 = jnp.zeros((16,), jnp.float32)

  @pl.loop(0, NPAD // 16)
  def _zero(i):
    degloc[pl.ds(i * 16, 16)] = zero16

  ones16 = jnp.ones((16,), jnp.float32)

  @pl.loop(0, NFULL // 2 - 1)
  def _chunk(g2):
    for b in (0, 1):
      c = g2 * 2 + b
      _load(c + 1, b ^ 1)
      _wait(c, b)
      for j in range(CH // 16):
        idx = dst2[b, pl.ds(j * 16, 16)]
        plsc.addupdate_scatter(degloc, [idx], ones16)

  _load(NFULL - 1, 1)
  for b in (0, 1):
    _wait(NFULL - 2 + b, b)
    for j in range(CH // 16):
      idx = dst2[b, pl.ds(j * 16, 16)]
      plsc.addupdate_scatter(degloc, [idx], ones16)

  offt = base + NFULL * CH
  pltpu.sync_copy(dst_hbm.at[pl.ds(offt, TAIL)], dst_t)
  plsc.addupdate_scatter(degloc, [dst_t[...]], ones16)

  pltpu.sync_copy(degloc, degp_hbm.at[pl.ds(wid * NPAD, NPAD)])


def _mm1_body(p_ref, degt_ref, x_ref, wl_ref, wr_ref, b_ref,
              z_ref, s_ref, ss_ref):
  s = p_ref[0] + p_ref[1]
  deg = jnp.sum(degt_ref[...], axis=1, keepdims=True)
  agg = s / jnp.maximum(deg, 1.0)
  z = (lax.dot_general(agg, wl_ref[...], (((1,), (1,)), ((), ())),
                       preferred_element_type=jnp.float32)
       + lax.dot_general(x_ref[...], wr_ref[...], (((1,), (1,)), ((), ())),
                         preferred_element_type=jnp.float32)
       + b_ref[...])
  z_ref[...] = z
  s_ref[...] = jnp.broadcast_to(jnp.sum(z, axis=0, keepdims=True), (8, D))
  ss_ref[...] = jnp.broadcast_to(jnp.sum(z * z, axis=0, keepdims=True), (8, D))


_mm1 = pl.pallas_call(
    _mm1_body,
    out_shape=[
        jax.ShapeDtypeStruct((N, D), jnp.float32),
        jax.ShapeDtypeStruct((8, D), jnp.float32),
        jax.ShapeDtypeStruct((8, D), jnp.float32),
    ],
)


def _bn_relu_body(z_ref, s_ref, ss_ref, g_ref, bt_ref, h_ref):
  mean = s_ref[0:1] * (1.0 / N)
  var = ss_ref[0:1] * (1.0 / N) - mean * mean
  inv = lax.rsqrt(var + 1e-5)
  h = (z_ref[...] - mean) * (inv * g_ref[...]) + bt_ref[...]
  h_ref[...] = jnp.maximum(h, 0.0)


_bn_relu = pl.pallas_call(
    _bn_relu_body,
    out_shape=jax.ShapeDtypeStruct((N, D), jnp.float32),
)


def _mm2_body(p_ref, degt_ref, h_ref, wl_ref, wr_ref, b_ref, o_ref):
  s = p_ref[0] + p_ref[1]
  deg = jnp.sum(degt_ref[...], axis=1, keepdims=True)
  agg = s / jnp.maximum(deg, 1.0)
  o_ref[...] = (lax.dot_general(agg, wl_ref[...], (((1,), (1,)), ((), ())),
                                preferred_element_type=jnp.float32)
                + lax.dot_general(h_ref[...], wr_ref[...], (((1,), (1,)), ((), ())),
                                  preferred_element_type=jnp.float32)
                + b_ref[...])


_mm2 = pl.pallas_call(
    _mm2_body,
    out_shape=jax.ShapeDtypeStruct((N, D), jnp.float32),
)


def kernel(x, edge_index, W1l, W1r, b1, gamma, beta, W2l, W2r, b2):
  src = edge_index[0].astype(jnp.int32)
  dst = edge_index[1].astype(jnp.int32)

  zrow = jnp.zeros((SLC, D), jnp.float32)

  degt = _deg_hist(dst).reshape(NW, NPAD)[:, :N].T
  part1 = _seg_sum(x, src, dst, zrow).reshape(NC, N, D)
  z, s, ss = _mm1(part1, degt, x, W1l, W1r, b1.reshape(1, D))
  h = _bn_relu(z, s, ss, gamma.reshape(1, D), beta.reshape(1, D))
  part2 = _seg_sum(h, src, dst, zrow).reshape(NC, N, D)
  out = _mm2(part2, degt, h, W2l, W2r, b2.reshape(1, D))
  return out
